# baseline (speedup 1.0000x reference)
import functools

import jax
import jax.numpy as jnp
from jax import lax
from jax.experimental import pallas as pl
from jax.experimental.pallas import tpu as pltpu
from jax.experimental.pallas import tpu_sc as plsc

NROWS = 1000000
BATCH = 16384
EMBED = 16
NS = 16
L = 16
NIDX = 128
CHUNKS = BATCH // NS // NIDX
IDX_ROWS = BATCH // NIDX
PT = BATCH // NS

_mesh = plsc.VectorSubcoreMesh(
    core_axis_name="c", subcore_axis_name="s", num_cores=1)


@functools.partial(
    pl.kernel,
    out_type=jax.ShapeDtypeStruct((IDX_ROWS, NIDX), jnp.float32),
    mesh=_mesh,
    compiler_params=pltpu.CompilerParams(use_tc_tiling_on_sc=False),
    scratch_types=[
        pltpu.VMEM((CHUNKS, NIDX), jnp.int32),
        pltpu.VMEM((CHUNKS, NIDX), jnp.int32),
        pltpu.VMEM((PT, EMBED), jnp.float32),
        pltpu.VMEM((PT, EMBED), jnp.float32),
        pltpu.VMEM((CHUNKS, NIDX), jnp.float32),
        pltpu.VMEM((CHUNKS, NIDX), jnp.float32),
        pltpu.VMEM((CHUNKS, NIDX), jnp.float32),
        pltpu.VMEM((L,), jnp.float32),
        pltpu.VMEM((NS, L), jnp.float32),
        pltpu.VMEM_SHARED((NS, L), jnp.float32),
        pltpu.SemaphoreType.DMA,
    ],
)
def _rec_kernel(idx_u_hbm, idx_p_hbm, uemb_hbm, ub_hbm, pemb_hbm, pb_hbm,
                out_hbm, iu, ip, uv, pv, ubv, pbv, outv, totv, partloc,
                shared, sem):
    sid = lax.axis_index("s")
    row0 = sid * CHUNKS

    pltpu.sync_copy(idx_u_hbm.at[pl.ds(row0, CHUNKS)], iu)
    pltpu.sync_copy(idx_p_hbm.at[pl.ds(row0, CHUNKS)], ip)

    cps = []
    for c in range(CHUNKS):
        dst = pl.ds(c * NIDX, NIDX)
        cps.append(pltpu.async_copy(
            uemb_hbm.at[iu.at[c]], uv.at[dst], sem))
        cps.append(pltpu.async_copy(
            pemb_hbm.at[ip.at[c]], pv.at[dst], sem))
        cps.append(pltpu.async_copy(ub_hbm.at[iu.at[c]], ubv.at[c], sem))
        cps.append(pltpu.async_copy(pb_hbm.at[ip.at[c]], pbv.at[c], sem))
    for cp in cps:
        cp.wait()

    def mac(i, accs):
        a0, a1, a2, a3 = accs
        r = i * 4
        a0 = a0 + uv[r, :] * pv[r, :]
        a1 = a1 + uv[r + 1, :] * pv[r + 1, :]
        a2 = a2 + uv[r + 2, :] * pv[r + 2, :]
        a3 = a3 + uv[r + 3, :] * pv[r + 3, :]
        return a0, a1, a2, a3

    z = jnp.zeros((L,), jnp.float32)
    a0, a1, a2, a3 = lax.fori_loop(0, PT // 4, mac, (z, z, z, z))
    totv[...] = (a0 + a1) + (a2 + a3)

    pltpu.sync_copy(totv, shared.at[sid])
    plsc.subcore_barrier()
    pltpu.sync_copy(shared, partloc)

    tot = jnp.zeros((L,), jnp.float32)
    for j in range(NS):
        tot = tot + partloc[j, :]

    lane = lax.iota(jnp.int32, L)
    dnums = lax.GatherDimensionNumbers(
        offset_dims=(), collapsed_slice_dims=(0,), start_index_map=(0,))
    for shift in (8, 4, 2, 1):
        rot = lax.rem(lane + shift, L)
        tot = tot + lax.gather(
            tot, rot[:, None], dnums, slice_sizes=(1,),
            mode=lax.GatherScatterMode.PROMISE_IN_BOUNDS)

    for c in range(CHUNKS):
        for g in range(NIDX // L):
            s = pl.ds(g * L, L)
            x = tot + ubv[c, s] + pbv[c, s]
            outv[c, s] = 1.0 / (1.0 + jnp.exp(-x))

    pltpu.sync_copy(outv, out_hbm.at[pl.ds(row0, CHUNKS)])


def kernel(inputs, user_embedding, user_bias, places_embedding, places_bias):
    idx_u = inputs[:, 0].reshape(IDX_ROWS, NIDX)
    idx_p = inputs[:, 1].reshape(IDX_ROWS, NIDX)
    ub = user_bias.reshape(-1)
    pb = places_bias.reshape(-1)
    out = _rec_kernel(idx_u, idx_p, user_embedding, ub, places_embedding, pb)
    return out.reshape(BATCH, 1)
